# Initial kernel scaffold; baseline (speedup 1.0000x reference)
import functools

import jax
import jax.numpy as jnp
import numpy as np
from jax import lax
from jax.experimental import pallas as pl
from jax.experimental.pallas import tpu as pltpu
from jax.experimental.pallas import tpu_sc as plsc

N_NODES = 40962
FEAT = 128
HALF = FEAT // 2
NUM_NEW = 3 * N_NODES - 6
N_IDX = 2 * NUM_NEW
N_OUT64 = 2 * N_NODES + N_IDX
TOP64 = 2 * N_NODES

NC, NS = 2, 16
NW = NC * NS

IDX_ROWS = N_IDX // 128
IDX_PER_W = IDX_ROWS // NW

TOP_PER_W = (TOP64 // (8 * NW)) * 8
TOP_REM = TOP64 - TOP_PER_W * NW
COPY_CHUNK = 512
N_COPY_CHUNKS = TOP_PER_W // COPY_CHUNK

_ZBLK = 1024


def _tc_body(x_ref, a_ref, z_ref):
    z_ref[...] = jnp.dot(x_ref[...], a_ref[...],
                         preferred_element_type=jnp.float32)


def _make_avg_matrix():
    a = np.zeros((FEAT, HALF), np.float32)
    for f in range(HALF):
        a[2 * f, f] = 0.5
        a[2 * f + 1, f] = 0.5
    return jnp.asarray(a)


def _compute_z(x):
    n_blk = (N_NODES + _ZBLK - 1) // _ZBLK
    return pl.pallas_call(
        _tc_body,
        grid=(n_blk,),
        in_specs=[
            pl.BlockSpec((_ZBLK, FEAT), lambda i: (i, 0)),
            pl.BlockSpec((FEAT, HALF), lambda i: (0, 0)),
        ],
        out_specs=pl.BlockSpec((_ZBLK, HALF), lambda i: (i, 0)),
        out_shape=jax.ShapeDtypeStruct((N_NODES, HALF), jnp.float32),
    )(x, _make_avg_matrix())


def _sc_body(x_hbm, z_hbm, idx_hbm, out_hbm, idx_v, rows_v, copy_v, sem):
    wid = lax.axis_index("s") * NC + lax.axis_index("c")

    for c in range(N_COPY_CHUNKS):
        base = wid * TOP_PER_W + c * COPY_CHUNK
        pltpu.sync_copy(x_hbm.at[pl.ds(base, COPY_CHUNK)], copy_v)
        pltpu.sync_copy(copy_v, out_hbm.at[pl.ds(base, COPY_CHUNK)])

    @pl.when(wid == NW - 1)
    def _():
        tail = TOP_PER_W * NW
        pltpu.sync_copy(x_hbm.at[pl.ds(tail, TOP_REM)],
                        copy_v.at[pl.ds(0, TOP_REM)])
        pltpu.sync_copy(copy_v.at[pl.ds(0, TOP_REM)],
                        out_hbm.at[pl.ds(tail, TOP_REM)])

    pltpu.sync_copy(idx_hbm.at[pl.ds(wid * IDX_PER_W, IDX_PER_W)], idx_v)

    def gather_one(g, carry):
        pltpu.async_copy(z_hbm.at[idx_v.at[g]], rows_v, sem).wait()
        dst = TOP64 + (wid * IDX_PER_W + g) * 128
        pltpu.sync_copy(rows_v, out_hbm.at[pl.ds(dst, 128)])
        return carry

    lax.fori_loop(0, IDX_PER_W, gather_one, 0)


@jax.jit
def _run(x, idx32):
    z = _compute_z(x)
    x64 = x.reshape(TOP64, HALF)
    idx2d = idx32.reshape(IDX_ROWS, 128)
    mesh = plsc.VectorSubcoreMesh(core_axis_name="c", subcore_axis_name="s",
                                  num_cores=NC, num_subcores=NS)
    out64 = pl.kernel(
        _sc_body,
        out_type=jax.ShapeDtypeStruct((N_OUT64, HALF), jnp.float32),
        mesh=mesh,
        scratch_types=[
            pltpu.VMEM((IDX_PER_W, 128), jnp.int32),
            pltpu.VMEM((128, HALF), jnp.float32),
            pltpu.VMEM((COPY_CHUNK, HALF), jnp.float32),
            pltpu.SemaphoreType.DMA,
        ],
    )(x64, z, idx2d)
    return out64.reshape(2 * N_NODES + NUM_NEW, FEAT)


def kernel(x, upsample_neighs_order):
    return _run(x, upsample_neighs_order.astype(jnp.int32))


# speedup vs baseline: 75.3748x; 75.3748x over previous
import functools

import jax
import jax.numpy as jnp
import numpy as np
from jax import lax
from jax.experimental import pallas as pl
from jax.experimental.pallas import tpu as pltpu
from jax.experimental.pallas import tpu_sc as plsc

N_NODES = 40962
FEAT = 128
HALF = FEAT // 2
NUM_NEW = 3 * N_NODES - 6
N_IDX = 2 * NUM_NEW
N_OUT64 = 2 * N_NODES + N_IDX
TOP64 = 2 * N_NODES

NC, NS = 2, 16
NW = NC * NS

IDX_ROWS = N_IDX // 128
IDX_PER_W = IDX_ROWS // NW

TOP_PER_W = (TOP64 // (8 * NW)) * 8
TOP_REM = TOP64 - TOP_PER_W * NW
COPY_CHUNK = 512
N_COPY_CHUNKS = TOP_PER_W // COPY_CHUNK

_ZBLK = 1024


def _tc_body(x_ref, a_ref, z_ref):
    z_ref[...] = jnp.dot(x_ref[...], a_ref[...],
                         precision=lax.Precision.HIGHEST,
                         preferred_element_type=jnp.float32)


def _make_avg_matrix():
    a = np.zeros((FEAT, HALF), np.float32)
    for f in range(HALF):
        a[2 * f, f] = 0.5
        a[2 * f + 1, f] = 0.5
    return jnp.asarray(a)


def _compute_z(x):
    n_blk = (N_NODES + _ZBLK - 1) // _ZBLK
    return pl.pallas_call(
        _tc_body,
        grid=(n_blk,),
        in_specs=[
            pl.BlockSpec((_ZBLK, FEAT), lambda i: (i, 0)),
            pl.BlockSpec((FEAT, HALF), lambda i: (0, 0)),
        ],
        out_specs=pl.BlockSpec((_ZBLK, HALF), lambda i: (i, 0)),
        out_shape=jax.ShapeDtypeStruct((N_NODES, HALF), jnp.float32),
    )(x, _make_avg_matrix())


def _sc_body(x_hbm, z_hbm, idx_hbm, out_hbm, idx_v, rows_v, copy_v, sem):
    wid = lax.axis_index("s") * NC + lax.axis_index("c")

    for c in range(N_COPY_CHUNKS):
        base = wid * TOP_PER_W + c * COPY_CHUNK
        pltpu.sync_copy(x_hbm.at[pl.ds(base, COPY_CHUNK)], copy_v)
        pltpu.sync_copy(copy_v, out_hbm.at[pl.ds(base, COPY_CHUNK)])

    @pl.when(wid == NW - 1)
    def _():
        tail = TOP_PER_W * NW
        pltpu.sync_copy(x_hbm.at[pl.ds(tail, TOP_REM)],
                        copy_v.at[pl.ds(0, TOP_REM)])
        pltpu.sync_copy(copy_v.at[pl.ds(0, TOP_REM)],
                        out_hbm.at[pl.ds(tail, TOP_REM)])

    pltpu.sync_copy(idx_hbm.at[pl.ds(wid * IDX_PER_W, IDX_PER_W)], idx_v)

    def gather_one(g, carry):
        pltpu.async_copy(z_hbm.at[idx_v.at[g]], rows_v, sem).wait()
        dst = TOP64 + (wid * IDX_PER_W + g) * 128
        pltpu.sync_copy(rows_v, out_hbm.at[pl.ds(dst, 128)])
        return carry

    lax.fori_loop(0, IDX_PER_W, gather_one, 0)


@jax.jit
def _run(x, idx32):
    z = _compute_z(x)
    x64 = x.reshape(TOP64, HALF)
    idx2d = idx32.reshape(IDX_ROWS, 128)
    mesh = plsc.VectorSubcoreMesh(core_axis_name="c", subcore_axis_name="s",
                                  num_cores=NC, num_subcores=NS)
    out64 = pl.kernel(
        _sc_body,
        out_type=jax.ShapeDtypeStruct((N_OUT64, HALF), jnp.float32),
        mesh=mesh,
        compiler_params=pltpu.CompilerParams(use_tc_tiling_on_sc=False),
        scratch_types=[
            pltpu.VMEM((IDX_PER_W, 128), jnp.int32),
            pltpu.VMEM((128, HALF), jnp.float32),
            pltpu.VMEM((COPY_CHUNK, HALF), jnp.float32),
            pltpu.SemaphoreType.DMA,
        ],
    )(x64, z, idx2d)
    return out64.reshape(N_NODES + NUM_NEW, FEAT)


def kernel(x, upsample_neighs_order):
    return _run(x, upsample_neighs_order.astype(jnp.int32))
